# baseline (speedup 1.0000x reference)
import functools

import jax
import jax.numpy as jnp
import numpy as np
from jax import lax
from jax.experimental import pallas as pl
from jax.experimental.pallas import tpu as pltpu
from jax.experimental.pallas import tpu_sc as plsc

VOCAB = 1000000
BATCH = 16384
SEQ = 50
SEQ_PAD = 52
D = 64
NLANE = 16
NCHUNK32 = D // 32

NC = 2
NS = 16
NW = NC * NS
ROWS_PER_W = BATCH // NW
ROWS_PER_CHUNK = 2
CHUNKS_PER_W = ROWS_PER_W // ROWS_PER_CHUNK
IDX_PER_CHUNK = ROWS_PER_CHUNK * SEQ_PAD
NBUF = 4

_HI_MASK = np.int32(np.uint32(0xFFFF0000).view(np.int32))

_PERM = np.concatenate([
    np.arange(0, 32, 2), np.arange(1, 32, 2),
    np.arange(32, 64, 2), np.arange(33, 64, 2)])


def _sc_pooled_sum(tok2d, emb_bf):
  mesh = plsc.VectorSubcoreMesh(
      core_axis_name="c", subcore_axis_name="s", num_cores=NC, num_subcores=NS)

  @functools.partial(
      pl.kernel,
      out_type=jax.ShapeDtypeStruct((BATCH, D), jnp.float32),
      mesh=mesh,
      scratch_types=[
          pltpu.VMEM((CHUNKS_PER_W, IDX_PER_CHUNK), jnp.int32),
          pltpu.VMEM((NBUF, IDX_PER_CHUNK), jnp.float32),
          pltpu.VMEM((ROWS_PER_W, D), jnp.float32),
      ] + [pltpu.SemaphoreType.DMA] * NBUF,
      compiler_params=pltpu.CompilerParams(
          use_tc_tiling_on_sc=False, needs_layout_passes=False),
  )
  def k(tok_hbm, emb_hbm, out_hbm, idx_v, stage_v, out_v, *sems):
    wid = lax.axis_index("s") * NC + lax.axis_index("c")
    pltpu.sync_copy(tok_hbm.at[pl.ds(wid * CHUNKS_PER_W, CHUNKS_PER_W)], idx_v)

    def fire(b, c):
      pltpu.async_copy(emb_hbm.at[0].at[idx_v.at[c]], stage_v.at[b], sems[b])

    def wait(b, c):
      pltpu.make_async_copy(emb_hbm.at[0].at[idx_v.at[c]], stage_v.at[b],
                            sems[b]).wait()

    def accum(b, c):
      acc = stage_v[b, pl.ds(0, NLANE)]
      for t in range(1, 6):
        acc = acc + stage_v[b, pl.ds(t * NLANE, NLANE)]
      out_v[c, pl.ds(0, NLANE)] = acc

    for b in range(NBUF):
      fire(b, b)

    def outer(g, carry):
      for b in range(NBUF):
        c = g * NBUF + b
        wait(b, c)
        accum(b, c)

        @pl.when(c + NBUF < CHUNKS_PER_W)
        def _():
          fire(b, c + NBUF)
      return carry

    lax.fori_loop(0, CHUNKS_PER_W // NBUF, outer, 0)
    pltpu.sync_copy(out_v, out_hbm.at[pl.ds(wid * ROWS_PER_W, ROWS_PER_W)])

  return k(tok2d, emb_bf)


def _tc_linear(pooled, wt_scaled, b):
  blk = 2048

  def body(x_ref, w_ref, b_ref, o_ref):
    o_ref[...] = jnp.dot(
        x_ref[...], w_ref[...],
        preferred_element_type=jnp.float32) + b_ref[...]

  return pl.pallas_call(
      body,
      grid=(BATCH // blk,),
      in_specs=[
          pl.BlockSpec((blk, D), lambda i: (i, 0)),
          pl.BlockSpec((D, D), lambda i: (0, 0)),
          pl.BlockSpec((1, D), lambda i: (0, 0)),
      ],
      out_specs=pl.BlockSpec((blk, D), lambda i: (i, 0)),
      out_shape=jax.ShapeDtypeStruct((BATCH, D), jnp.float32),
  )(pooled, wt_scaled, b.reshape(1, D))


@jax.jit
def kernel(token_ids, emb, W, b):
  tok_pad = jnp.pad(token_ids.astype(jnp.int32), ((0, 0), (0, SEQ_PAD - SEQ)))
  tok2d = tok_pad.reshape(BATCH // ROWS_PER_CHUNK, IDX_PER_CHUNK)
  pooled = _sc_pooled_sum(tok2d, emb.T)
  wt_scaled = (W.T * (1.0 / SEQ))[jnp.asarray(_PERM), :]
  return _tc_linear(pooled, wt_scaled, b)


# speedup vs baseline: 2.8755x; 2.8755x over previous
import functools

import jax
import jax.numpy as jnp
import numpy as np
from jax import lax
from jax.experimental import pallas as pl
from jax.experimental.pallas import tpu as pltpu
from jax.experimental.pallas import tpu_sc as plsc

VOCAB = 1000000
BATCH = 16384
SEQ = 50
SEQ_PAD = 52
D = 64
DH = 32
NLANE = 16

NC = 2
NS = 16
NW = NC * NS
ROWS_PER_W = BATCH // NW
ROWS_PER_CHUNK = 2
CHUNKS_PER_W = ROWS_PER_W // ROWS_PER_CHUNK
IDX_PER_CHUNK = ROWS_PER_CHUNK * SEQ_PAD
NBUF = 4

_HI_MASK = np.int32(np.uint32(0xFFFF0000).view(np.int32))

_PERM = np.concatenate([
    np.arange(0, 32, 2), np.arange(1, 32, 2),
    np.arange(32, 64, 2), np.arange(33, 64, 2)])


def _sc_half_pooled(tok2d, emb_bf):
  mesh = plsc.VectorSubcoreMesh(
      core_axis_name="c", subcore_axis_name="s", num_cores=NC, num_subcores=NS)

  @functools.partial(
      pl.kernel,
      out_type=jax.ShapeDtypeStruct((BATCH, DH), jnp.float32),
      mesh=mesh,
      scratch_types=[
          pltpu.VMEM((CHUNKS_PER_W, IDX_PER_CHUNK), jnp.int32),
          pltpu.VMEM((NBUF, IDX_PER_CHUNK, DH), jnp.bfloat16),
          pltpu.VMEM((ROWS_PER_W, DH), jnp.float32),
      ] + [pltpu.SemaphoreType.DMA] * NBUF,
      compiler_params=pltpu.CompilerParams(
          use_tc_tiling_on_sc=False, needs_layout_passes=False),
  )
  def k(tok_hbm, emb_hbm, out_hbm, idx_v, stage_v, out_v, *sems):
    wid = lax.axis_index("s") * NC + lax.axis_index("c")
    pltpu.sync_copy(tok_hbm.at[pl.ds(wid * CHUNKS_PER_W, CHUNKS_PER_W)], idx_v)

    def fire(b, c):
      pltpu.async_copy(emb_hbm.at[idx_v.at[c]], stage_v.at[b], sems[b])

    def wait(b, c):
      pltpu.make_async_copy(emb_hbm.at[idx_v.at[c]], stage_v.at[b],
                            sems[b]).wait()

    def row_unpacked(b, r):
      packed = plsc.bitcast(stage_v[b, r, pl.ds(0, 32)], jnp.int32)
      even = plsc.bitcast(packed << 16, jnp.float32)
      odd = plsc.bitcast(packed & _HI_MASK, jnp.float32)
      return [even, odd]

    def accum(b, c):
      def row_body(j, carry):
        base = j * SEQ_PAD
        accs = row_unpacked(b, base)
        for t in range(1, SEQ):
          vals = row_unpacked(b, base + t)
          for d in range(2):
            accs[d] = accs[d] + vals[d]
        for d in range(2):
          out_v[ROWS_PER_CHUNK * c + j, pl.ds(NLANE * d, NLANE)] = accs[d]
        return carry

      lax.fori_loop(0, ROWS_PER_CHUNK, row_body, 0)

    for b in range(NBUF):
      fire(b, b)

    def outer(g, carry):
      for b in range(NBUF):
        c = g * NBUF + b
        wait(b, c)
        accum(b, c)

        @pl.when(c + NBUF < CHUNKS_PER_W)
        def _():
          fire(b, c + NBUF)
      return carry

    lax.fori_loop(0, CHUNKS_PER_W // NBUF, outer, 0)
    pltpu.sync_copy(out_v, out_hbm.at[pl.ds(wid * ROWS_PER_W, ROWS_PER_W)])

  return k(tok2d, emb_bf)


def _tc_linear(pooled, wt_scaled, b):
  blk = 2048

  def body(x_ref, w_ref, b_ref, o_ref):
    o_ref[...] = jnp.dot(
        x_ref[...], w_ref[...],
        preferred_element_type=jnp.float32) + b_ref[...]

  return pl.pallas_call(
      body,
      grid=(BATCH // blk,),
      in_specs=[
          pl.BlockSpec((blk, D), lambda i: (i, 0)),
          pl.BlockSpec((D, D), lambda i: (0, 0)),
          pl.BlockSpec((1, D), lambda i: (0, 0)),
      ],
      out_specs=pl.BlockSpec((blk, D), lambda i: (i, 0)),
      out_shape=jax.ShapeDtypeStruct((BATCH, D), jnp.float32),
  )(pooled, wt_scaled, b.reshape(1, D))


@jax.jit
def kernel(token_ids, emb, W, b):
  tok_pad = jnp.pad(token_ids.astype(jnp.int32), ((0, 0), (0, SEQ_PAD - SEQ)))
  tok2d = tok_pad.reshape(BATCH // ROWS_PER_CHUNK, IDX_PER_CHUNK)
  emb_a = emb[:, :DH].astype(jnp.bfloat16)
  emb_b = emb[:, DH:].astype(jnp.bfloat16)
  pooled_a = _sc_half_pooled(tok2d, emb_a)
  pooled_b = _sc_half_pooled(tok2d, emb_b)
  pooled = jnp.concatenate([pooled_a, pooled_b], axis=1)
  wt_scaled = (W.T * (1.0 / SEQ))[jnp.asarray(_PERM), :]
  return _tc_linear(pooled, wt_scaled, b)


# speedup vs baseline: 4.2525x; 1.4789x over previous
import functools

import jax
import jax.numpy as jnp
import numpy as np
from jax import lax
from jax.experimental import pallas as pl
from jax.experimental.pallas import tpu as pltpu
from jax.experimental.pallas import tpu_sc as plsc

VOCAB = 1000000
BATCH = 16384
SEQ = 50
SEQ_PAD = 52
D = 64
NLANE = 16
NCHUNK32 = D // 32

NC = 2
NS = 16
NW = NC * NS
ROWS_PER_W = BATCH // NW
ROWS_PER_CHUNK = 2
CHUNKS_PER_W = ROWS_PER_W // ROWS_PER_CHUNK
IDX_PER_CHUNK = ROWS_PER_CHUNK * SEQ_PAD
NBUF = 4

_HI_MASK = np.int32(np.uint32(0xFFFF0000).view(np.int32))

_PERM = np.concatenate([
    np.arange(0, 32, 2), np.arange(1, 32, 2),
    np.arange(32, 64, 2), np.arange(33, 64, 2)])


def _sc_pooled_sum(tok2d, emb_bf):
  mesh = plsc.VectorSubcoreMesh(
      core_axis_name="c", subcore_axis_name="s", num_cores=NC, num_subcores=NS)

  @functools.partial(
      pl.kernel,
      out_type=jax.ShapeDtypeStruct((BATCH, D), jnp.float32),
      mesh=mesh,
      scratch_types=[
          pltpu.VMEM((CHUNKS_PER_W, IDX_PER_CHUNK), jnp.int32),
          pltpu.VMEM((NBUF, IDX_PER_CHUNK, D), jnp.bfloat16),
          pltpu.VMEM((ROWS_PER_W, D), jnp.float32),
      ] + [pltpu.SemaphoreType.DMA] * NBUF,
      compiler_params=pltpu.CompilerParams(
          use_tc_tiling_on_sc=False, needs_layout_passes=False),
  )
  def k(tok_hbm, emb_hbm, out_hbm, idx_v, stage_v, out_v, *sems):
    wid = lax.axis_index("s") * NC + lax.axis_index("c")
    pltpu.sync_copy(tok_hbm.at[pl.ds(wid * CHUNKS_PER_W, CHUNKS_PER_W)], idx_v)

    def fire(b, c):
      pltpu.async_copy(emb_hbm.at[idx_v.at[c]], stage_v.at[b], sems[b])

    def wait(b, c):
      pltpu.make_async_copy(emb_hbm.at[idx_v.at[c]], stage_v.at[b],
                            sems[b]).wait()

    def row_unpacked(b, r):
      out = []
      for kk in range(NCHUNK32):
        packed = plsc.bitcast(stage_v[b, r, pl.ds(32 * kk, 32)], jnp.int32)
        even = plsc.bitcast(packed << 16, jnp.float32)
        odd = plsc.bitcast(packed & _HI_MASK, jnp.float32)
        out += [even, odd]
      return out

    def accum(b, c):
      def row_body(j, carry):
        base = j * SEQ_PAD
        accs = row_unpacked(b, base)
        for t in range(1, SEQ):
          vals = row_unpacked(b, base + t)
          for d in range(4):
            accs[d] = accs[d] + vals[d]
        for d in range(4):
          out_v[ROWS_PER_CHUNK * c + j, pl.ds(NLANE * d, NLANE)] = accs[d]
        return carry

      lax.fori_loop(0, ROWS_PER_CHUNK, row_body, 0)

    for b in range(NBUF):
      fire(b, b)

    def outer(g, carry):
      for b in range(NBUF):
        c = g * NBUF + b
        wait(b, c)
        accum(b, c)

        @pl.when(c + NBUF < CHUNKS_PER_W)
        def _():
          fire(b, c + NBUF)
      return carry

    lax.fori_loop(0, CHUNKS_PER_W // NBUF, outer, 0)
    pltpu.sync_copy(out_v, out_hbm.at[pl.ds(wid * ROWS_PER_W, ROWS_PER_W)])

  return k(tok2d, emb_bf)


def _tc_linear(pooled, wt_scaled, b):
  blk = 2048

  def body(x_ref, w_ref, b_ref, o_ref):
    o_ref[...] = jnp.dot(
        x_ref[...], w_ref[...],
        preferred_element_type=jnp.float32) + b_ref[...]

  return pl.pallas_call(
      body,
      grid=(BATCH // blk,),
      in_specs=[
          pl.BlockSpec((blk, D), lambda i: (i, 0)),
          pl.BlockSpec((D, D), lambda i: (0, 0)),
          pl.BlockSpec((1, D), lambda i: (0, 0)),
      ],
      out_specs=pl.BlockSpec((blk, D), lambda i: (i, 0)),
      out_shape=jax.ShapeDtypeStruct((BATCH, D), jnp.float32),
  )(pooled, wt_scaled, b.reshape(1, D))


@jax.jit
def kernel(token_ids, emb, W, b):
  tok_pad = jnp.pad(token_ids.astype(jnp.int32), ((0, 0), (0, SEQ_PAD - SEQ)))
  tok2d = tok_pad.reshape(BATCH // ROWS_PER_CHUNK, IDX_PER_CHUNK)
  emb_bf = emb.astype(jnp.bfloat16)
  pooled = _sc_pooled_sum(tok2d, emb_bf)
  wt_scaled = (W.T * (1.0 / SEQ))[jnp.asarray(_PERM), :]
  return _tc_linear(pooled, wt_scaled, b)
